# Initial kernel scaffold; baseline (speedup 1.0000x reference)
import jax
import jax.numpy as jnp
from jax.experimental import pallas as pl


def kernel(x, W):
    raise NotImplementedError("write your pallas kernel here")


# speedup vs baseline: 4.9540x; 4.9540x over previous
import functools

import jax
import jax.numpy as jnp
from jax import lax
from jax.experimental import pallas as pl
from jax.experimental.pallas import tpu as pltpu
from jax.experimental.pallas import tpu_sc as plsc

NUM_UNITS = 1000000
NUM_PHONEMES = 32
BATCH = 16384
HIST = 200

NW = 32
G = 128
K = 8
ROWS_PER_GROUP = K * G
TOTAL = BATCH * HIST
GROUPS = TOTAL // ROWS_PER_GROUP
GROUPS_PER_W = GROUPS // NW


def _gather_kernel(x_hbm, w_hbm, out_hbm, idx_buf, rows, gsem0, gsem1,
                   ssem0, ssem1):
    wid = lax.axis_index("s") * 2 + lax.axis_index("c")
    g0 = wid * GROUPS_PER_W
    gsem = (gsem0, gsem1)
    ssem = (ssem0, ssem1)

    def fire_gathers(p, u):
        del u
        for j in range(K):
            pltpu.async_copy(w_hbm.at[idx_buf.at[p, j]], rows.at[p, j],
                             gsem[p])

    def drain_gathers(p):
        for j in range(K):
            pltpu.make_async_copy(w_hbm.at[idx_buf.at[p, j]], rows.at[p, j],
                                  gsem[p]).wait()

    def fire_store(p, u):
        pltpu.async_copy(rows.at[p], out_hbm.at[u], ssem[p])

    def drain_store(p, u):
        pltpu.make_async_copy(rows.at[p], out_hbm.at[u], ssem[p]).wait()

    def load_idx(p, u):
        pltpu.sync_copy(x_hbm.at[u], idx_buf.at[p])

    def body(u, p, drain_prev_store, process_prev):
        q = 1 - p
        if drain_prev_store:
            drain_store(p, u - 2)
        load_idx(p, u)
        fire_gathers(p, u)
        if process_prev:
            drain_gathers(q)
            fire_store(q, u - 1)

    body(g0, 0, False, False)
    body(g0 + 1, 1, False, True)

    def loop_body(k, _):
        u = g0 + 2 * k
        body(u, 0, True, True)
        body(u + 1, 1, True, True)
        return _

    lax.fori_loop(1, GROUPS_PER_W // 2, loop_body, None)

    last = g0 + GROUPS_PER_W - 1
    drain_gathers(1)
    fire_store(1, last)
    drain_store(0, last - 1)
    drain_store(1, last)


@jax.jit
def _run(x_flat, w):
    mesh = plsc.VectorSubcoreMesh(core_axis_name="c", subcore_axis_name="s")
    out = pl.kernel(
        _gather_kernel,
        out_type=jax.ShapeDtypeStruct((GROUPS, K, G, NUM_PHONEMES),
                                      jnp.float32),
        mesh=mesh,
        scratch_types=[
            pltpu.VMEM((2, K, G), jnp.int32),
            pltpu.VMEM((2, K, G, NUM_PHONEMES), jnp.float32),
            pltpu.SemaphoreType.DMA,
            pltpu.SemaphoreType.DMA,
            pltpu.SemaphoreType.DMA,
            pltpu.SemaphoreType.DMA,
        ],
        compiler_params=pltpu.CompilerParams(use_tc_tiling_on_sc=False),
    )(x_flat, w)
    return out.reshape(BATCH, HIST, NUM_PHONEMES)


def kernel(x, W):
    x_flat = x.astype(jnp.int32).reshape(GROUPS, K, G)
    return _run(x_flat, W)
